# baseline (speedup 1.0000x reference)
import functools

import jax
import jax.numpy as jnp
from jax import lax
from jax.experimental import pallas as pl
from jax.experimental.pallas import tpu as pltpu
from jax.experimental.pallas import tpu_sc as plsc

NBUF = 5
NC = 2
NS = 16
NW = NC * NS
L = 16
SEQ = 50
BBLK = 128


def _embed_body(seq_hbm, table_hbm, out_hbm, idx_v, idx2, gbufs, tbufs,
                *sems):
    gsems = sems[:NBUF]
    osems = sems[NBUF:]
    c = lax.axis_index("c")
    s_ax = lax.axis_index("s")
    bt = s_ax * NC + c

    pltpu.sync_copy(seq_hbm.at[:, pl.ds(bt * BBLK, BBLK)], idx_v)

    iota = lax.iota(jnp.int32, L)

    def build(s, _):
        for k in range(8):
            v2 = idx_v[s, pl.ds(k * L, L)] * 2
            pos = (k % 4) * (2 * L) + iota * 2
            row = idx2.at[2 * s + (k // 4)]
            plsc.store_scatter(row, [pos], v2)
            plsc.store_scatter(row, [pos + 1], v2 + 1)
        return 0

    lax.fori_loop(0, SEQ, build, 0)

    def gathers(s, slot):
        lo = pltpu.make_async_copy(
            table_hbm.at[idx2.at[2 * s]],
            gbufs.at[slot, pl.ds(0, BBLK)], gsems[slot])
        hi = pltpu.make_async_copy(
            table_hbm.at[idx2.at[2 * s + 1]],
            gbufs.at[slot, pl.ds(BBLK, BBLK)], gsems[slot])
        return lo, hi

    for slot in range(NBUF):
        for cp in gathers(slot, slot):
            cp.start()

    rows = [(iota + k * L) * 2 for k in range(8)]
    zeros = iota * 0

    def transpose(slot):
        def one_d(i, _):
            vals = []
            for u in range(4):
                d = i * 4 + u
                rhalf = lax.shift_right_logical(d, 5)
                col = zeros + lax.bitwise_and(d, 31)
                vals.append([
                    plsc.load_gather(gbufs.at[slot], [rows[k] + rhalf, col])
                    for k in range(8)
                ])
            for u in range(4):
                d = i * 4 + u
                dt = lax.shift_right_logical(d, 3)
                din = lax.bitwise_and(d, 7)
                for k in range(8):
                    tbufs[slot, dt, 0, din, pl.ds(k * L, L)] = vals[u][k]
            return 0

        lax.fori_loop(0, 16, one_d, 0)

    def one_round(i, refill):
        writes = []
        for slot in range(NBUF):
            s = i * NBUF + slot
            lo, hi = gathers(s, slot)
            lo.wait()
            hi.wait()
            transpose(slot)
            writes.append(
                pltpu.async_copy(
                    tbufs.at[slot],
                    out_hbm.at[s, pl.ds(0, 8), pl.ds(bt, 1)],
                    osems[slot],
                )
            )
        for slot in range(NBUF):
            writes[slot].wait()
            if refill:
                for cp in gathers((i + 1) * NBUF + slot, slot):
                    cp.start()

    n_rounds = SEQ // NBUF
    lax.fori_loop(0, n_rounds - 1, lambda i, _: (one_round(i, True), 0)[1], 0)
    one_round(n_rounds - 1, False)


@jax.jit
def _embed_call(seq_t, table2):
    grid_kernel = pl.kernel(
        _embed_body,
        out_type=jax.ShapeDtypeStruct((SEQ, 8, NW, 8, BBLK), jnp.float32),
        mesh=plsc.VectorSubcoreMesh(
            core_axis_name="c", subcore_axis_name="s",
            num_cores=NC, num_subcores=NS,
        ),
        scratch_types=[
            pltpu.VMEM((SEQ, BBLK), jnp.int32),
            pltpu.VMEM((2 * SEQ, BBLK), jnp.int32),
            pltpu.VMEM((NBUF, 2 * BBLK, 32), jnp.float32),
            pltpu.VMEM((NBUF, 8, 1, 8, BBLK), jnp.float32),
        ] + [pltpu.SemaphoreType.DMA] * (2 * NBUF),
        compiler_params=pltpu.CompilerParams(
            use_tc_tiling_on_sc=False, needs_layout_passes=False
        ),
    )
    return grid_kernel(seq_t, table2)


def kernel(seq, table):
    batch, seq_len = seq.shape
    assert (batch, seq_len) == (NW * BBLK, SEQ)
    seq_t = jnp.transpose(seq).astype(jnp.int32)
    table2 = table.reshape(table.shape[0] * 2, 32)
    out5 = _embed_call(seq_t, table2)
    return jnp.transpose(out5, (2, 4, 0, 1, 3)).reshape(
        batch, seq_len, table.shape[1])


# speedup vs baseline: 1.1277x; 1.1277x over previous
import functools

import jax
import jax.numpy as jnp
from jax import lax
from jax.experimental import pallas as pl
from jax.experimental.pallas import tpu as pltpu
from jax.experimental.pallas import tpu_sc as plsc

NBUF = 10
CHUNK = 64
NC = 2
NS = 16
NW = NC * NS
L = 16


def _embed_body(lpw, seq_hbm, table_hbm, out_hbm, idx_v, idx2, bufs, *sems):
    gsems = sems[:NBUF]
    osems = sems[NBUF:]
    c = lax.axis_index("c")
    s = lax.axis_index("s")
    wid = s * NC + c

    pltpu.sync_copy(seq_hbm.at[wid], idx_v)

    iota = lax.iota(jnp.int32, L)

    def build(g, _):
        v2 = idx_v[g] * 2
        pos = g * (2 * L) + iota * 2
        plsc.store_scatter(idx2, [pos], v2)
        plsc.store_scatter(idx2, [pos + 1], v2 + 1)
        return 0

    lax.fori_loop(0, lpw // L, build, 0)

    def gather(j, slot):
        return pltpu.make_async_copy(
            table_hbm.at[idx2.at[pl.ds(j * 2 * CHUNK, 2 * CHUNK)]],
            bufs.at[slot],
            gsems[slot],
        )

    for slot in range(NBUF):
        gather(slot, slot).start()

    out0 = wid * 2 * lpw

    def one_round(i, refill):
        writes = []
        for slot in range(NBUF):
            j = i * NBUF + slot
            gather(j, slot).wait()
            writes.append(
                pltpu.async_copy(
                    bufs.at[slot],
                    out_hbm.at[pl.ds(out0 + j * 2 * CHUNK, 2 * CHUNK)],
                    osems[slot],
                )
            )
        for slot in range(NBUF):
            writes[slot].wait()
            if refill:
                gather((i + 1) * NBUF + slot, slot).start()

    n_rounds = lpw // (CHUNK * NBUF)
    lax.fori_loop(0, n_rounds - 1, lambda i, _: (one_round(i, True), 0)[1], 0)
    one_round(n_rounds - 1, False)


@jax.jit
def _embed_call(seq3d, table2):
    nw, rows, lanes = seq3d.shape
    lpw = rows * lanes
    grid_kernel = pl.kernel(
        functools.partial(_embed_body, lpw),
        out_type=jax.ShapeDtypeStruct((NW * lpw * 2, 32), jnp.float32),
        mesh=plsc.VectorSubcoreMesh(
            core_axis_name="c", subcore_axis_name="s",
            num_cores=NC, num_subcores=NS,
        ),
        scratch_types=[
            pltpu.VMEM((rows, lanes), jnp.int32),
            pltpu.VMEM((lpw * 2,), jnp.int32),
            pltpu.VMEM((NBUF, 2 * CHUNK, 32), jnp.float32),
        ] + [pltpu.SemaphoreType.DMA] * (2 * NBUF),
        compiler_params=pltpu.CompilerParams(
            use_tc_tiling_on_sc=False, needs_layout_passes=False
        ),
    )
    return grid_kernel(seq3d, table2)


def kernel(seq, table):
    batch, seq_len = seq.shape
    total = batch * seq_len
    assert total % (NW * CHUNK * NBUF) == 0
    seq3d = seq.reshape(NW, total // (NW * L), L).astype(jnp.int32)
    table2 = table.reshape(table.shape[0] * 2, 32)
    out = _embed_call(seq3d, table2)
    return out.reshape(batch, seq_len, table.shape[1])
